# Initial kernel scaffold; baseline (speedup 1.0000x reference)
import functools

import jax
import jax.numpy as jnp
from jax import lax
from jax.experimental import pallas as pl
from jax.experimental.pallas import tpu as pltpu
from jax.experimental.pallas import tpu_sc as plsc

NC = 2
NS = 16
NW = NC * NS
LANES = 16


def _mesh():
    return plsc.VectorSubcoreMesh(
        core_axis_name="c", subcore_axis_name="s",
        num_cores=NC, num_subcores=NS)


@functools.lru_cache(maxsize=None)
def _make_prep(nobj, npred, emb, f_rows):
    fpw = f_rows // NW
    opw = nobj // NW
    ochunk = 625
    nchunks = opw // ochunk
    assert fpw % LANES == 0 and opw % ochunk == 0 and emb % LANES == 0
    ec = emb // LANES

    @functools.partial(
        pl.kernel,
        out_type=[jax.ShapeDtypeStruct((f_rows, emb), jnp.float32),
                  jax.ShapeDtypeStruct((nobj, emb), jnp.float32)],
        mesh=_mesh(),
        scratch_types=[
            pltpu.VMEM((fpw,), jnp.int32),
            pltpu.VMEM((fpw, emb), jnp.float32),
            pltpu.VMEM((2, emb), jnp.float32),
            pltpu.VMEM((ochunk, emb), jnp.float32),
            pltpu.SemaphoreType.DMA,
        ],
    )
    def prep(bool_hbm, pred_hbm, obj_hbm, f_hbm, r_hbm,
             idx_v, prow, bbuf, rbuf, sem):
        wid = lax.axis_index("s") * NC + lax.axis_index("c")
        base = wid * fpw

        for g in range(fpw // LANES):
            r = base + g * LANES + lax.iota(jnp.int32, (LANES,))
            idx_v[pl.ds(g * LANES, LANES)] = lax.rem(r, npred)
        pltpu.async_copy(pred_hbm.at[idx_v], prow, sem).wait()
        pltpu.sync_copy(bool_hbm, bbuf)
        b0 = [jnp.maximum(bbuf[0, pl.ds(c * LANES, LANES)], 0.0)
              for c in range(ec)]
        b1 = [jnp.maximum(bbuf[1, pl.ds(c * LANES, LANES)], 0.0)
              for c in range(ec)]

        def frow(j, carry):
            v = (base + j) >= npred
            for c in range(ec):
                s = pl.ds(c * LANES, LANES)
                pr = jnp.maximum(prow[j, s], 0.0)
                prow[j, s] = pr * jnp.where(v, b1[c], b0[c])
            return carry
        lax.fori_loop(0, fpw, frow, 0)
        pltpu.sync_copy(prow, f_hbm.at[pl.ds(base, fpw)])

        def relu_chunk(k, carry):
            rbase = wid * opw + k * ochunk
            pltpu.sync_copy(obj_hbm.at[pl.ds(rbase, ochunk)], rbuf)

            def rrow(j, c2):
                for c in range(ec):
                    s = pl.ds(c * LANES, LANES)
                    rbuf[j, s] = jnp.maximum(rbuf[j, s], 0.0)
                return c2
            lax.fori_loop(0, ochunk, rrow, 0)
            pltpu.sync_copy(rbuf, r_hbm.at[pl.ds(rbase, ochunk)])
            return carry
        lax.fori_loop(0, nchunks, relu_chunk, 0)

    return prep


@functools.lru_cache(maxsize=None)
def _make_main(n, emb, f_rows, nobj, npred):
    per_w = n // NW
    C = 128
    nchunks = per_w // C
    assert n % NW == 0 and per_w % C == 0

    @functools.partial(
        pl.kernel,
        out_type=[jax.ShapeDtypeStruct((n, emb), jnp.float32),
                  jax.ShapeDtypeStruct((n, emb), jnp.float32)],
        mesh=_mesh(),
        scratch_types=[
            pltpu.VMEM((C,), jnp.int32),
            pltpu.VMEM((C,), jnp.int32),
            pltpu.VMEM((C,), jnp.int32),
            pltpu.VMEM((C,), jnp.int32),
            pltpu.VMEM((C, emb), jnp.float32),
            pltpu.VMEM((C, emb), jnp.float32),
            pltpu.SemaphoreType.DMA,
            pltpu.SemaphoreType.DMA,
        ],
    )
    def mainc(vv_hbm, vt_hbm, oc_hbm, f_hbm, r_hbm, hp_hbm, ho_hbm,
              vvb, vtb, ocb, fib, hpb, hob, sem1, sem2):
        wid = lax.axis_index("s") * NC + lax.axis_index("c")
        base = wid * per_w

        def chunk(k, carry):
            off = base + k * C
            pltpu.sync_copy(vv_hbm.at[pl.ds(off, C)], vvb)
            pltpu.sync_copy(vt_hbm.at[pl.ds(off, C)], vtb)
            pltpu.sync_copy(oc_hbm.at[pl.ds(off, C)], ocb)
            for g in range(C // LANES):
                s = pl.ds(g * LANES, LANES)
                fib[s] = vvb[s] * npred + vtb[s]
            cp1 = pltpu.async_copy(f_hbm.at[fib], hpb, sem1)
            cp2 = pltpu.async_copy(r_hbm.at[ocb], hob, sem2)
            cp1.wait()
            cp2.wait()
            pltpu.sync_copy(hpb, hp_hbm.at[pl.ds(off, C)])
            pltpu.sync_copy(hob, ho_hbm.at[pl.ds(off, C)])
            return carry
        lax.fori_loop(0, nchunks, chunk, 0)

    return mainc


def kernel(var_val, var_type, object_class, bool_table, pred_table, obj_table):
    b, l = var_val.shape
    nobj, emb = obj_table.shape
    npred = pred_table.shape[0]
    n = b * l
    f_rows = 2048

    vv = var_val.reshape(n).astype(jnp.int32)
    vt = var_type.reshape(n).astype(jnp.int32)
    oc = object_class.reshape(n).astype(jnp.int32)

    f_tab, r_tab = _make_prep(nobj, npred, emb, f_rows)(
        bool_table, pred_table, obj_table)
    hp, ho = _make_main(n, emb, f_rows, nobj, npred)(vv, vt, oc, f_tab, r_tab)
    return hp.reshape(b, l, emb), ho.reshape(b, l, emb)


# speedup vs baseline: 7.1671x; 7.1671x over previous
import functools

import jax
import jax.numpy as jnp
from jax import lax
from jax.experimental import pallas as pl
from jax.experimental.pallas import tpu as pltpu
from jax.experimental.pallas import tpu_sc as plsc

NC = 2
NS = 16
NW = NC * NS
LANES = 16


def _mesh():
    return plsc.VectorSubcoreMesh(
        core_axis_name="c", subcore_axis_name="s",
        num_cores=NC, num_subcores=NS)


@functools.lru_cache(maxsize=None)
def _make_prep(nobj, npred, emb, f_rows):
    fpw = f_rows // NW
    ochunk = 800
    nchunks = nobj // ochunk
    nit = -(-nchunks // NW)
    assert fpw % LANES == 0 and nobj % ochunk == 0 and emb % LANES == 0
    ec = emb // LANES

    @functools.partial(
        pl.kernel,
        out_type=[jax.ShapeDtypeStruct((f_rows, emb), jnp.float32),
                  jax.ShapeDtypeStruct((nobj, emb), jnp.float32)],
        mesh=_mesh(),
        compiler_params=pltpu.CompilerParams(use_tc_tiling_on_sc=False),
        scratch_types=[
            pltpu.VMEM((fpw,), jnp.int32),
            pltpu.VMEM((fpw, emb), jnp.float32),
            pltpu.VMEM((2, emb), jnp.float32),
            pltpu.VMEM((800, emb), jnp.float32),
            pltpu.SemaphoreType.DMA,
        ],
    )
    def prep(bool_hbm, pred_hbm, obj_hbm, f_hbm, r_hbm,
             idx_v, prow, bbuf, rbuf, sem):
        wid = lax.axis_index("s") * NC + lax.axis_index("c")
        base = wid * fpw

        for g in range(fpw // LANES):
            r = base + g * LANES + lax.iota(jnp.int32, LANES)
            idx_v[pl.ds(g * LANES, LANES)] = lax.rem(r, npred)
        pltpu.async_copy(pred_hbm.at[idx_v], prow, sem).wait()
        pltpu.sync_copy(bool_hbm, bbuf)
        b0 = [jnp.maximum(bbuf[0, pl.ds(c * LANES, LANES)], 0.0)
              for c in range(ec)]
        b1 = [jnp.maximum(bbuf[1, pl.ds(c * LANES, LANES)], 0.0)
              for c in range(ec)]

        def frow(j, carry):
            v = (base + j) >= npred
            for c in range(ec):
                s = pl.ds(c * LANES, LANES)
                pr = jnp.maximum(prow[j, s], 0.0)
                prow[j, s] = pr * jnp.where(v, b1[c], b0[c])
            return carry
        lax.fori_loop(0, fpw, frow, 0)
        pltpu.sync_copy(prow, f_hbm.at[pl.ds(base, fpw)])

        def relu_chunk(t, carry):
            cid = wid + t * NW

            @pl.when(cid < nchunks)
            def _():
                rbase = cid * ochunk
                pltpu.sync_copy(obj_hbm.at[pl.ds(rbase, ochunk)], rbuf)

                def rrow(j, c2):
                    for c in range(ec):
                        s = pl.ds(c * LANES, LANES)
                        rbuf[j, s] = jnp.maximum(rbuf[j, s], 0.0)
                    return c2
                lax.fori_loop(0, ochunk, rrow, 0)
                pltpu.sync_copy(rbuf, r_hbm.at[pl.ds(rbase, ochunk)])
            return carry
        lax.fori_loop(0, nit, relu_chunk, 0)

    return prep


@functools.lru_cache(maxsize=None)
def _make_main(n, emb, f_rows, nobj, npred):
    per_w = n // NW
    C = 128
    nchunks = per_w // C
    assert n % NW == 0 and per_w % C == 0

    @functools.partial(
        pl.kernel,
        out_type=[jax.ShapeDtypeStruct((n, emb), jnp.float32),
                  jax.ShapeDtypeStruct((n, emb), jnp.float32)],
        mesh=_mesh(),
        compiler_params=pltpu.CompilerParams(use_tc_tiling_on_sc=False),
        scratch_types=[
            pltpu.VMEM((C,), jnp.int32),
            pltpu.VMEM((C,), jnp.int32),
            pltpu.VMEM((C,), jnp.int32),
            pltpu.VMEM((C,), jnp.int32),
            pltpu.VMEM((C, emb), jnp.float32),
            pltpu.VMEM((C, emb), jnp.float32),
            pltpu.SemaphoreType.DMA,
            pltpu.SemaphoreType.DMA,
        ],
    )
    def mainc(vv_hbm, vt_hbm, oc_hbm, f_hbm, r_hbm, hp_hbm, ho_hbm,
              vvb, vtb, ocb, fib, hpb, hob, sem1, sem2):
        wid = lax.axis_index("s") * NC + lax.axis_index("c")
        base = wid * per_w

        def chunk(k, carry):
            off = base + k * C
            pltpu.sync_copy(vv_hbm.at[pl.ds(off, C)], vvb)
            pltpu.sync_copy(vt_hbm.at[pl.ds(off, C)], vtb)
            pltpu.sync_copy(oc_hbm.at[pl.ds(off, C)], ocb)
            for g in range(C // LANES):
                s = pl.ds(g * LANES, LANES)
                fib[s] = vvb[s] * npred + vtb[s]
            cp1 = pltpu.async_copy(f_hbm.at[fib], hpb, sem1)
            cp2 = pltpu.async_copy(r_hbm.at[ocb], hob, sem2)
            cp1.wait()
            cp2.wait()
            pltpu.sync_copy(hpb, hp_hbm.at[pl.ds(off, C)])
            pltpu.sync_copy(hob, ho_hbm.at[pl.ds(off, C)])
            return carry
        lax.fori_loop(0, nchunks, chunk, 0)

    return mainc


def kernel(var_val, var_type, object_class, bool_table, pred_table, obj_table):
    b, l = var_val.shape
    nobj, emb = obj_table.shape
    npred = pred_table.shape[0]
    n = b * l
    f_rows = 2048

    vv = var_val.reshape(n).astype(jnp.int32)
    vt = var_type.reshape(n).astype(jnp.int32)
    oc = object_class.reshape(n).astype(jnp.int32)

    f_tab, r_tab = _make_prep(nobj, npred, emb, f_rows)(
        bool_table, pred_table, obj_table)
    hp, ho = _make_main(n, emb, f_rows, nobj, npred)(vv, vt, oc, f_tab, r_tab)
    return hp.reshape(b, l, emb), ho.reshape(b, l, emb)


# speedup vs baseline: 9.1204x; 1.2726x over previous
import functools

import jax
import jax.numpy as jnp
from jax import lax
from jax.experimental import pallas as pl
from jax.experimental.pallas import tpu as pltpu
from jax.experimental.pallas import tpu_sc as plsc

NC = 2
NS = 16
NW = NC * NS
LANES = 16


def _mesh():
    return plsc.VectorSubcoreMesh(
        core_axis_name="c", subcore_axis_name="s",
        num_cores=NC, num_subcores=NS)


@functools.lru_cache(maxsize=None)
def _make_prep(nobj, npred, emb, f_rows):
    fpw = f_rows // NW
    ochunk = 800
    nchunks = nobj // ochunk
    nit = -(-nchunks // NW)
    assert fpw % LANES == 0 and nobj % ochunk == 0 and emb % LANES == 0
    ec = emb // LANES

    @functools.partial(
        pl.kernel,
        out_type=[jax.ShapeDtypeStruct((f_rows, emb), jnp.float32),
                  jax.ShapeDtypeStruct((nobj, emb), jnp.float32)],
        mesh=_mesh(),
        compiler_params=pltpu.CompilerParams(use_tc_tiling_on_sc=False),
        scratch_types=[
            pltpu.VMEM((fpw,), jnp.int32),
            pltpu.VMEM((fpw, emb), jnp.float32),
            pltpu.VMEM((2, emb), jnp.float32),
            pltpu.VMEM((800, emb), jnp.float32),
            pltpu.SemaphoreType.DMA,
        ],
    )
    def prep(bool_hbm, pred_hbm, obj_hbm, f_hbm, r_hbm,
             idx_v, prow, bbuf, rbuf, sem):
        wid = lax.axis_index("s") * NC + lax.axis_index("c")
        base = wid * fpw

        for g in range(fpw // LANES):
            r = base + g * LANES + lax.iota(jnp.int32, LANES)
            idx_v[pl.ds(g * LANES, LANES)] = lax.rem(r, npred)
        pltpu.async_copy(pred_hbm.at[idx_v], prow, sem).wait()
        pltpu.sync_copy(bool_hbm, bbuf)
        b0 = [jnp.maximum(bbuf[0, pl.ds(c * LANES, LANES)], 0.0)
              for c in range(ec)]
        b1 = [jnp.maximum(bbuf[1, pl.ds(c * LANES, LANES)], 0.0)
              for c in range(ec)]

        def frow(j, carry):
            v = (base + j) >= npred
            for c in range(ec):
                s = pl.ds(c * LANES, LANES)
                pr = jnp.maximum(prow[j, s], 0.0)
                prow[j, s] = pr * jnp.where(v, b1[c], b0[c])
            return carry
        lax.fori_loop(0, fpw, frow, 0)
        pltpu.sync_copy(prow, f_hbm.at[pl.ds(base, fpw)])

        def relu_chunk(t, carry):
            cid = wid + t * NW

            @pl.when(cid < nchunks)
            def _():
                rbase = cid * ochunk
                pltpu.sync_copy(obj_hbm.at[pl.ds(rbase, ochunk)], rbuf)

                def rrow(j, c2):
                    for c in range(ec):
                        s = pl.ds(c * LANES, LANES)
                        rbuf[j, s] = jnp.maximum(rbuf[j, s], 0.0)
                    return c2
                lax.fori_loop(0, ochunk, rrow, 0)
                pltpu.sync_copy(rbuf, r_hbm.at[pl.ds(rbase, ochunk)])
            return carry
        lax.fori_loop(0, nit, relu_chunk, 0)

    return prep


@functools.lru_cache(maxsize=None)
def _make_main(n, emb, f_rows, nobj, npred):
    per_w = n // NW
    C = 256
    CR = C // 128
    M = per_w // C
    assert n % NW == 0 and per_w % C == 0 and M % 2 == 0

    @functools.partial(
        pl.kernel,
        out_type=[jax.ShapeDtypeStruct((n, emb), jnp.float32),
                  jax.ShapeDtypeStruct((n, emb), jnp.float32)],
        mesh=_mesh(),
        compiler_params=pltpu.CompilerParams(use_tc_tiling_on_sc=False),
        scratch_types=[
            pltpu.VMEM((2, C), jnp.int32),
            pltpu.VMEM((2, C), jnp.int32),
            pltpu.VMEM((2, C), jnp.int32),
            pltpu.VMEM((2, C), jnp.int32),
            pltpu.VMEM((2, C), jnp.int32),
            pltpu.VMEM((2, C, emb), jnp.float32),
            pltpu.VMEM((2, C, emb), jnp.float32),
            pltpu.SemaphoreType.DMA,
            pltpu.SemaphoreType.DMA,
            pltpu.SemaphoreType.DMA,
            pltpu.SemaphoreType.DMA,
            pltpu.SemaphoreType.DMA,
            pltpu.SemaphoreType.DMA,
        ],
    )
    def mainc(vv_hbm, vt_hbm, oc_hbm, f_hbm, r_hbm, hp_hbm, ho_hbm,
              vvb, vtb, ocb1, fib, ocb, hpb, hob,
              is0, is1, gs0, gs1, ws0, ws1):
        wid = lax.axis_index("s") * NC + lax.axis_index("c")
        base = wid * per_w
        isem = (is0, is1)
        gsem = (gs0, gs1)
        wsem = (ws0, ws1)

        def idx_descs(c, b):
            off = base + c * C
            return [
                pltpu.make_async_copy(vv_hbm.at[pl.ds(off, C)], vvb.at[b],
                                      isem[b]),
                pltpu.make_async_copy(vt_hbm.at[pl.ds(off, C)], vtb.at[b],
                                      isem[b]),
                pltpu.make_async_copy(oc_hbm.at[pl.ds(off, C)], ocb1.at[b],
                                      isem[b]),
            ]

        def gat_descs(b):
            return [
                pltpu.make_async_copy(f_hbm.at[fib.at[b]], hpb.at[b],
                                      gsem[b]),
                pltpu.make_async_copy(r_hbm.at[ocb.at[b]], hob.at[b],
                                      gsem[b]),
            ]

        def wr_descs(c, b):
            off = base + c * C
            return [
                pltpu.make_async_copy(hpb.at[b], hp_hbm.at[pl.ds(off, C)],
                                      wsem[b]),
                pltpu.make_async_copy(hob.at[b], ho_hbm.at[pl.ds(off, C)],
                                      wsem[b]),
            ]

        for cc in (0, 1):
            for d in idx_descs(cc, cc):
                d.start()

        def body(i, carry):
            for b in (0, 1):
                c = 2 * i + b
                for d in idx_descs(c, b):
                    d.wait()
                for g in range(C // LANES):
                    s = pl.ds(g * LANES, LANES)
                    fib[b, s] = vvb[b, s] * npred + vtb[b, s]
                    ocb[b, s] = ocb1[b, s]

                @pl.when(c + 2 <= M - 1)
                def _():
                    for d in idx_descs(c + 2, b):
                        d.start()

                @pl.when(c >= 2)
                def _():
                    for d in wr_descs(c - 2, b):
                        d.wait()

                for d in gat_descs(b):
                    d.start()

                @pl.when(c >= 1)
                def _():
                    for d in gat_descs(b ^ 1):
                        d.wait()
                    for d in wr_descs(c - 1, b ^ 1):
                        d.start()
            return carry
        lax.fori_loop(0, M // 2, body, 0)

        bl = (M - 1) % 2
        for d in gat_descs(bl):
            d.wait()
        for d in wr_descs(M - 1, bl):
            d.start()
        for d in wr_descs(M - 2, bl ^ 1):
            d.wait()
        for d in wr_descs(M - 1, bl):
            d.wait()

    return mainc


def kernel(var_val, var_type, object_class, bool_table, pred_table, obj_table):
    b, l = var_val.shape
    nobj, emb = obj_table.shape
    npred = pred_table.shape[0]
    n = b * l
    f_rows = 2048

    vv = var_val.reshape(n).astype(jnp.int32)
    vt = var_type.reshape(n).astype(jnp.int32)
    oc = object_class.reshape(n).astype(jnp.int32)

    f_tab, r_tab = _make_prep(nobj, npred, emb, f_rows)(
        bool_table, pred_table, obj_table)
    hp, ho = _make_main(n, emb, f_rows, nobj, npred)(vv, vt, oc, f_tab, r_tab)
    return hp.reshape(b, l, emb), ho.reshape(b, l, emb)
